# baseline (speedup 1.0000x reference)
import functools
import math

import jax
import jax.numpy as jnp
from jax import lax
from jax.experimental import pallas as pl
from jax.experimental.pallas import tpu as pltpu
from jax.experimental.pallas import tpu_sc as plsc

LANES = 16
NC = 2
NS = 16
NW = NC * NS
IDX_B = 128


def _mm_body(xt_ref, w_ref, o_ref):
    @pl.when(pl.program_id(1) == 0)
    def _():
        o_ref[...] = jnp.zeros_like(o_ref)

    o_ref[...] += lax.dot_general(
        w_ref[...], xt_ref[...],
        dimension_numbers=(((1,), (0,)), ((), ())),
        preferred_element_type=jnp.float32,
    )


def _matmul(w, xt_pad, ko, nk, bj=128):
    ne = w.shape[0]
    bk = ne // nk
    return pl.pallas_call(
        _mm_body,
        grid=(ne // bj, 1),
        in_specs=[
            pl.BlockSpec((bk, LANES), lambda j, k: (ko, 0)),
            pl.BlockSpec((bj, bk), lambda j, k: (j, ko)),
        ],
        out_specs=pl.BlockSpec((bj, LANES), lambda j, k: (j, 0)),
        out_shape=jax.ShapeDtypeStruct((ne, LANES), jnp.float32),
        compiler_params=pltpu.CompilerParams(
            dimension_semantics=("parallel", "arbitrary")),
    )(xt_pad, w)


def _make_spmm(ne, nb):
    rows_per_tile = ne // NS
    mesh = plsc.VectorSubcoreMesh(
        core_axis_name="c", subcore_axis_name="s",
        num_cores=NC, num_subcores=NS)

    @functools.partial(
        pl.kernel,
        mesh=mesh,
        compiler_params=pltpu.CompilerParams(use_tc_tiling_on_sc=False),
        out_type=jax.ShapeDtypeStruct((NC * ne, LANES), jnp.float32),
        scratch_types=[
            pltpu.VMEM((nb, IDX_B), jnp.int32),
            pltpu.VMEM((nb, IDX_B), jnp.int32),
            pltpu.VMEM((nb, IDX_B), jnp.float32),
            pltpu.VMEM((IDX_B, LANES), jnp.float32),
            pltpu.VMEM((IDX_B, LANES), jnp.float32),
            pltpu.VMEM((IDX_B, LANES), jnp.float32),
            pltpu.VMEM((IDX_B, LANES), jnp.float32),
            pltpu.VMEM_SHARED((ne, LANES), jnp.float32),
            pltpu.SemaphoreType.DMA,
            pltpu.SemaphoreType.DMA,
            pltpu.SemaphoreType.DMA,
            pltpu.SemaphoreType.DMA,
            pltpu.SemaphoreType.DMA,
        ],
    )
    def spmm(h_hbm, rows_hbm, cols_hbm, w_hbm, init_hbm, out_hbm,
             rows_v, cols_v, w_v, gbufa, gbufb, cbufa, cbufb, acc,
             stsem, gsema, gsemb, ssema, ssemb):
        c = lax.axis_index("c")
        s = lax.axis_index("s")
        wid = c * NS + s
        nh = nb // 2

        pltpu.async_copy(rows_hbm.at[wid], rows_v, stsem)
        pltpu.async_copy(cols_hbm.at[wid], cols_v, stsem)
        pltpu.async_copy(w_hbm.at[wid], w_v, stsem)
        init_slice = init_hbm.at[
            pl.ds(c * ne + s * rows_per_tile, rows_per_tile)]
        acc_slice = acc.at[pl.ds(s * rows_per_tile, rows_per_tile)]
        pltpu.async_copy(init_slice, acc_slice, stsem)

        pltpu.make_async_copy(rows_hbm.at[wid], rows_v, stsem).wait()
        pltpu.make_async_copy(cols_hbm.at[wid], cols_v, stsem).wait()
        pltpu.make_async_copy(w_hbm.at[wid], w_v, stsem).wait()
        pltpu.make_async_copy(init_slice, acc_slice, stsem).wait()
        plsc.subcore_barrier()

        def _compute(gb, cb, b):
            for g in range(IDX_B // LANES):
                wvec = w_v[b, pl.ds(g * LANES, LANES)]
                for j in range(LANES):
                    i = g * LANES + j
                    cb[i] = gb[i] * wvec[j]

        pltpu.async_copy(h_hbm.at[rows_v.at[0]], gbufa, gsema)

        def _pipe(g, carry):
            b0 = 2 * g
            b1 = b0 + 1
            pltpu.async_copy(h_hbm.at[rows_v.at[b1]], gbufb, gsemb)
            pltpu.make_async_copy(h_hbm.at[rows_v.at[0]], gbufa, gsema).wait()

            @pl.when(g > 0)
            def _():
                pltpu.make_async_copy(cbufa, acc.at[cols_v.at[0]], ssema).wait()
            _compute(gbufa, cbufa, b0)
            pltpu.async_copy(cbufa, acc.at[cols_v.at[b0]], ssema, add=True)

            @pl.when(g + 1 < nh)
            def _():
                pltpu.async_copy(h_hbm.at[rows_v.at[b0 + 2]], gbufa, gsema)
            pltpu.make_async_copy(h_hbm.at[rows_v.at[0]], gbufb, gsemb).wait()

            @pl.when(g > 0)
            def _():
                pltpu.make_async_copy(cbufb, acc.at[cols_v.at[0]], ssemb).wait()
            _compute(gbufb, cbufb, b1)
            pltpu.async_copy(cbufb, acc.at[cols_v.at[b1]], ssemb, add=True)
            return carry
        lax.fori_loop(0, nh, _pipe, 0)

        pltpu.make_async_copy(cbufa, acc.at[cols_v.at[0]], ssema).wait()
        pltpu.make_async_copy(cbufb, acc.at[cols_v.at[0]], ssemb).wait()

        plsc.subcore_barrier()
        pltpu.sync_copy(
            acc.at[pl.ds(s * rows_per_tile, rows_per_tile)],
            out_hbm.at[pl.ds(c * ne + s * rows_per_tile, rows_per_tile)])

    return spmm


def _make_combine(ne, b, bj=2048):
    def _body(p_ref, o_ref):
        t = p_ref[0] + p_ref[1]
        t = jnp.where(t >= 0, t, jnp.float32(0.001) * t)
        eye = jnp.eye(b, dtype=jnp.float32)
        o_ref[...] = lax.dot_general(
            eye, t[:, :b],
            dimension_numbers=(((1,), (1,)), ((), ())),
            preferred_element_type=jnp.float32,
        )

    return pl.pallas_call(
        _body,
        grid=(ne // bj,),
        in_specs=[pl.BlockSpec((2, bj, LANES), lambda j: (0, j, 0))],
        out_specs=pl.BlockSpec((b, bj), lambda j: (0, j)),
        out_shape=jax.ShapeDtypeStruct((b, ne), jnp.float32),
    )


def kernel(x, fc1_weight, sparse_weights, rows, cols):
    b, ne = x.shape
    nnz = rows.shape[0]

    xt_pad = jnp.zeros((ne, LANES), jnp.float32).at[:, :b].set(x.T)
    h_a = _matmul(fc1_weight, xt_pad, 0, 2)
    h_b = _matmul(fc1_weight, xt_pad, 1, 2)

    nb = 2 * math.ceil(nnz / (NW * IDX_B * 2))
    total = NW * nb * IDX_B
    pad = total - nnz
    rows_p = jnp.pad(rows.astype(jnp.int32), (0, pad)).reshape(NW, nb, IDX_B)
    cols_p = jnp.pad(cols.astype(jnp.int32), (0, pad)).reshape(NW, nb, IDX_B)
    w_p = jnp.pad(sparse_weights, (0, pad)).reshape(NW, nb, IDX_B)

    spmm = _make_spmm(ne, nb)
    zeros = jnp.zeros((NC * ne, LANES), jnp.float32)
    parts_a = spmm(h_a, rows_p, cols_p, w_p, zeros)
    parts = spmm(h_b, rows_p, cols_p, w_p, parts_a)
    out = _make_combine(ne, b)(parts.reshape(2, ne, LANES))
    return out


# speedup vs baseline: 1.2605x; 1.2605x over previous
import functools
import math

import jax
import jax.numpy as jnp
from jax import lax
from jax.experimental import pallas as pl
from jax.experimental.pallas import tpu as pltpu
from jax.experimental.pallas import tpu_sc as plsc

LANES = 16
NC = 2
NS = 16
NW = NC * NS
IDX_B = 128


def _mm_body(xt_ref, w_ref, o_ref):
    @pl.when(pl.program_id(1) == 0)
    def _():
        o_ref[...] = jnp.zeros_like(o_ref)

    o_ref[...] += lax.dot_general(
        w_ref[...], xt_ref[...],
        dimension_numbers=(((1,), (0,)), ((), ())),
        preferred_element_type=jnp.float32,
    )


def _matmul(w, xt_pad, ko, nk, bj=128):
    ne = w.shape[0]
    bk = ne // nk
    return pl.pallas_call(
        _mm_body,
        grid=(ne // bj, 1),
        in_specs=[
            pl.BlockSpec((bk, LANES), lambda j, k: (ko, 0)),
            pl.BlockSpec((bj, bk), lambda j, k: (j, ko)),
        ],
        out_specs=pl.BlockSpec((bj, LANES), lambda j, k: (j, 0)),
        out_shape=jax.ShapeDtypeStruct((ne, LANES), jnp.float32),
        compiler_params=pltpu.CompilerParams(
            dimension_semantics=("parallel", "arbitrary")),
    )(xt_pad, w)


def _make_spmm(ne, nb):
    rows_per_tile = ne // NS
    mesh = plsc.VectorSubcoreMesh(
        core_axis_name="c", subcore_axis_name="s",
        num_cores=NC, num_subcores=NS)

    @functools.partial(
        pl.kernel,
        mesh=mesh,
        compiler_params=pltpu.CompilerParams(use_tc_tiling_on_sc=False),
        out_type=jax.ShapeDtypeStruct((NC * ne, LANES), jnp.float32),
        scratch_types=[
            pltpu.VMEM((nb, IDX_B), jnp.int32),
            pltpu.VMEM((nb, IDX_B), jnp.int32),
            pltpu.VMEM((nb, IDX_B), jnp.float32),
            pltpu.VMEM((IDX_B, LANES), jnp.float32),
            pltpu.VMEM((IDX_B, LANES), jnp.float32),
            pltpu.VMEM((IDX_B, LANES), jnp.float32),
            pltpu.VMEM((IDX_B, LANES), jnp.float32),
            pltpu.VMEM_SHARED((ne, LANES), jnp.float32),
            pltpu.SemaphoreType.DMA,
            pltpu.SemaphoreType.DMA,
            pltpu.SemaphoreType.DMA,
            pltpu.SemaphoreType.DMA,
            pltpu.SemaphoreType.DMA,
        ],
    )
    def spmm(h_hbm, rows_hbm, cols_hbm, w_hbm, init_hbm, out_hbm,
             rows_v, cols_v, w_v, gbufa, gbufb, cbufa, cbufb, acc,
             stsem, gsema, gsemb, ssema, ssemb):
        c = lax.axis_index("c")
        s = lax.axis_index("s")
        wid = c * NS + s
        nh = nb // 2

        pltpu.async_copy(rows_hbm.at[wid], rows_v, stsem)
        pltpu.async_copy(cols_hbm.at[wid], cols_v, stsem)
        pltpu.async_copy(w_hbm.at[wid], w_v, stsem)
        init_slice = init_hbm.at[
            pl.ds(c * ne + s * rows_per_tile, rows_per_tile)]
        acc_slice = acc.at[pl.ds(s * rows_per_tile, rows_per_tile)]
        pltpu.async_copy(init_slice, acc_slice, stsem)

        pltpu.make_async_copy(rows_hbm.at[wid], rows_v, stsem).wait()
        pltpu.make_async_copy(cols_hbm.at[wid], cols_v, stsem).wait()
        pltpu.make_async_copy(w_hbm.at[wid], w_v, stsem).wait()
        pltpu.make_async_copy(init_slice, acc_slice, stsem).wait()
        plsc.subcore_barrier()

        def _compute(gb, cb, b):
            for g in range(IDX_B // LANES):
                wvec = w_v[b, pl.ds(g * LANES, LANES)]
                for j in range(LANES):
                    i = g * LANES + j
                    cb[i] = gb[i] * wvec[j]

        pltpu.async_copy(h_hbm.at[rows_v.at[0]], gbufa, gsema)

        def _pipe(g, carry):
            b0 = 2 * g
            b1 = b0 + 1
            pltpu.async_copy(h_hbm.at[rows_v.at[b1]], gbufb, gsemb)
            pltpu.make_async_copy(h_hbm.at[rows_v.at[0]], gbufa, gsema).wait()

            @pl.when(g > 0)
            def _():
                pltpu.make_async_copy(cbufa, acc.at[cols_v.at[0]], ssema).wait()
            _compute(gbufa, cbufa, b0)
            pltpu.async_copy(cbufa, acc.at[cols_v.at[b0]], ssema, add=True)

            @pl.when(g + 1 < nh)
            def _():
                pltpu.async_copy(h_hbm.at[rows_v.at[b0 + 2]], gbufa, gsema)
            pltpu.make_async_copy(h_hbm.at[rows_v.at[0]], gbufb, gsemb).wait()

            @pl.when(g > 0)
            def _():
                pltpu.make_async_copy(cbufb, acc.at[cols_v.at[0]], ssemb).wait()
            _compute(gbufb, cbufb, b1)
            pltpu.async_copy(cbufb, acc.at[cols_v.at[b1]], ssemb, add=True)
            return carry
        lax.fori_loop(0, nh, _pipe, 0)

        pltpu.make_async_copy(cbufa, acc.at[cols_v.at[0]], ssema).wait()
        pltpu.make_async_copy(cbufb, acc.at[cols_v.at[0]], ssemb).wait()

        plsc.subcore_barrier()
        pltpu.sync_copy(
            acc.at[pl.ds(s * rows_per_tile, rows_per_tile)],
            out_hbm.at[pl.ds(c * ne + s * rows_per_tile, rows_per_tile)])

    return spmm


def _make_combine(ne, b, bj=2048):
    def _body(p_ref, o_ref):
        t = p_ref[0] + p_ref[1]
        t = jnp.where(t >= 0, t, jnp.float32(0.001) * t)
        eye = jnp.eye(b, dtype=jnp.float32)
        o_ref[...] = lax.dot_general(
            eye, t[:, :b],
            dimension_numbers=(((1,), (1,)), ((), ())),
            preferred_element_type=jnp.float32,
        )

    return pl.pallas_call(
        _body,
        grid=(ne // bj,),
        in_specs=[pl.BlockSpec((2, bj, LANES), lambda j: (0, j, 0))],
        out_specs=pl.BlockSpec((b, bj), lambda j: (0, j)),
        out_shape=jax.ShapeDtypeStruct((b, ne), jnp.float32),
    )


def kernel(x, fc1_weight, sparse_weights, rows, cols):
    b, ne = x.shape
    nnz = rows.shape[0]

    xt_pad = jnp.zeros((ne, LANES), jnp.float32).at[:, :b].set(x.T)
    h = _matmul(fc1_weight, xt_pad, 0, 1)

    nb = 2 * math.ceil(nnz / (NW * IDX_B * 2))
    total = NW * nb * IDX_B
    pad = total - nnz
    rows_p = jnp.pad(rows.astype(jnp.int32), (0, pad)).reshape(NW, nb, IDX_B)
    cols_p = jnp.pad(cols.astype(jnp.int32), (0, pad)).reshape(NW, nb, IDX_B)
    w_p = jnp.pad(sparse_weights, (0, pad)).reshape(NW, nb, IDX_B)

    zeros = jnp.zeros((NC * ne, LANES), jnp.float32)
    parts = _make_spmm(ne, nb)(h, rows_p, cols_p, w_p, zeros)
    out = _make_combine(ne, b)(parts.reshape(2, ne, LANES))
    return out
